# baseline (speedup 1.0000x reference)
import functools

import jax
import jax.numpy as jnp
from jax import lax
from jax.experimental import pallas as pl
from jax.experimental.pallas import tpu as pltpu
from jax.experimental.pallas import tpu_sc as plsc

_NUM_CORES = 2
_NUM_SUBCORES = 16
_NW = _NUM_CORES * _NUM_SUBCORES
_LANES = 16
_IDX_CHUNK = 128


@functools.partial(jax.jit, static_argnums=(2, 3, 4))
def _mean_embed_sc(idx_flat, table, B, C, D):
    rows_per_w = (B // _NW) * C
    b_per_w = B // _NW
    n_chunks = rows_per_w // _IDX_CHUNK
    V = table.shape[0]
    table_pad = jnp.concatenate(
        [table, jnp.zeros((V, 128 - D), jnp.float32)], axis=1)
    DP = table_pad.shape[1]
    idx3 = idx_flat.reshape(_NW, n_chunks, _IDX_CHUNK)
    mesh = plsc.VectorSubcoreMesh(
        core_axis_name="c", subcore_axis_name="s",
        num_cores=_NUM_CORES, num_subcores=_NUM_SUBCORES)

    @functools.partial(
        pl.kernel,
        mesh=mesh,
        out_type=jax.ShapeDtypeStruct((B, D), jnp.float32),
        scratch_types=[
            pltpu.VMEM((n_chunks, _IDX_CHUNK), jnp.int32),
            pltpu.VMEM((rows_per_w, DP), jnp.float32),
            pltpu.VMEM((b_per_w, D), jnp.float32),
            pltpu.SemaphoreType.DMA,
        ],
    )
    def sc_kernel(idx_hbm, table_hbm, out_hbm, idx_v, rows_v, out_v, sem):
        wid = lax.axis_index("s") * _NUM_CORES + lax.axis_index("c")
        pltpu.sync_copy(idx_hbm.at[wid], idx_v)
        copies = []
        for k in range(n_chunks):
            copies.append(pltpu.async_copy(
                table_hbm.at[idx_v.at[k]],
                rows_v.at[pl.ds(k * _IDX_CHUNK, _IDX_CHUNK)],
                sem))
        for c in copies:
            c.wait()

        inv = jnp.float32(1.0 / C)

        def body(b, carry):
            r0 = b * C
            for d in range(D // _LANES):
                sl = pl.ds(d * _LANES, _LANES)
                acc = rows_v[r0, sl]
                for j in range(1, C):
                    acc = acc + rows_v[r0 + j, sl]
                out_v[b, sl] = acc * inv
            return carry

        lax.fori_loop(0, b_per_w, body, 0)
        pltpu.sync_copy(out_v, out_hbm.at[pl.ds(wid * b_per_w, b_per_w)])

    return sc_kernel(idx3, table_pad)


def _mm_body(wt_ref, mean_ref, b_ref, out_ref):
    acc = lax.dot_general(
        wt_ref[...], mean_ref[...].astype(jnp.bfloat16),
        (((0,), (1,)), ((), ())),
        preferred_element_type=jnp.float32)
    out_ref[...] = acc + jnp.transpose(b_ref[...])


@functools.partial(jax.jit, static_argnums=(3,))
def _project_tc(mean, Wt, b, block_v):
    B, D = mean.shape
    V = Wt.shape[1]
    n_blocks = pl.cdiv(V, block_v)
    out_t = pl.pallas_call(
        _mm_body,
        grid=(n_blocks,),
        in_specs=[
            pl.BlockSpec((D, block_v), lambda i: (0, i)),
            pl.BlockSpec((B, D), lambda i: (0, 0)),
            pl.BlockSpec((1, block_v), lambda i: (0, i)),
        ],
        out_specs=pl.BlockSpec((block_v, B), lambda i: (i, 0)),
        out_shape=jax.ShapeDtypeStruct((V, B), jnp.float32),
    )(Wt, mean, b.reshape(1, V))
    return out_t.T


def kernel(context_words, embedding, W, b):
    B, C = context_words.shape
    V, D = embedding.shape
    idx_flat = context_words.reshape(B * C).astype(jnp.int32)
    mean = _mean_embed_sc(idx_flat, embedding, B, C, D)
    return _project_tc(mean, W.T.astype(jnp.bfloat16), b, 4096)


# speedup vs baseline: 1.0230x; 1.0230x over previous
import functools

import jax
import jax.numpy as jnp
from jax import lax
from jax.experimental import pallas as pl
from jax.experimental.pallas import tpu as pltpu
from jax.experimental.pallas import tpu_sc as plsc

_NUM_CORES = 2
_NUM_SUBCORES = 16
_NW = _NUM_CORES * _NUM_SUBCORES
_LANES = 16
_IDX_CHUNK = 128


@functools.partial(jax.jit, static_argnums=(2, 3, 4))
def _mean_embed_sc(idx_flat, table, B, C, D):
    rows_per_w = (B // _NW) * C
    b_per_w = B // _NW
    n_chunks = rows_per_w // _IDX_CHUNK
    V = table.shape[0]
    table_pad = jnp.pad(table.T, ((0, 128 - D), (0, 0))).T
    DP = table_pad.shape[1]
    idx3 = idx_flat.reshape(_NW, n_chunks, _IDX_CHUNK)
    mesh = plsc.VectorSubcoreMesh(
        core_axis_name="c", subcore_axis_name="s",
        num_cores=_NUM_CORES, num_subcores=_NUM_SUBCORES)

    @functools.partial(
        pl.kernel,
        mesh=mesh,
        out_type=jax.ShapeDtypeStruct((B, D), jnp.float32),
        scratch_types=[
            pltpu.VMEM((n_chunks, _IDX_CHUNK), jnp.int32),
            pltpu.VMEM((rows_per_w, DP), jnp.float32),
            pltpu.VMEM((b_per_w, D), jnp.float32),
            pltpu.SemaphoreType.DMA,
        ],
    )
    def sc_kernel(idx_hbm, table_hbm, out_hbm, idx_v, rows_v, out_v, sem):
        wid = lax.axis_index("s") * _NUM_CORES + lax.axis_index("c")
        pltpu.sync_copy(idx_hbm.at[wid], idx_v)
        copies = []
        for k in range(n_chunks):
            copies.append(pltpu.async_copy(
                table_hbm.at[idx_v.at[k]],
                rows_v.at[pl.ds(k * _IDX_CHUNK, _IDX_CHUNK)],
                sem))
        for c in copies:
            c.wait()

        inv = jnp.float32(1.0 / C)

        def body(b, carry):
            r0 = b * C
            for d in range(D // _LANES):
                sl = pl.ds(d * _LANES, _LANES)
                acc = rows_v[r0, sl]
                for j in range(1, C):
                    acc = acc + rows_v[r0 + j, sl]
                out_v[b, sl] = acc * inv
            return carry

        lax.fori_loop(0, b_per_w, body, 0)
        pltpu.sync_copy(out_v, out_hbm.at[pl.ds(wid * b_per_w, b_per_w)])

    return sc_kernel(idx3, table_pad)


def _mm_body(wt_ref, mean_ref, b_ref, out_ref):
    acc = lax.dot_general(
        wt_ref[...], mean_ref[...],
        (((0,), (1,)), ((), ())),
        preferred_element_type=jnp.float32)
    out_ref[...] = acc + jnp.transpose(b_ref[...])


@functools.partial(jax.jit, static_argnums=(3,))
def _project_tc(mean, Wt, b, block_v):
    B, D = mean.shape
    V = Wt.shape[1]
    n_blocks = pl.cdiv(V, block_v)
    out_t = pl.pallas_call(
        _mm_body,
        grid=(n_blocks,),
        in_specs=[
            pl.BlockSpec((D, block_v), lambda i: (0, i)),
            pl.BlockSpec((B, D), lambda i: (0, 0)),
            pl.BlockSpec((1, block_v), lambda i: (0, i)),
        ],
        out_specs=pl.BlockSpec((block_v, B), lambda i: (i, 0)),
        out_shape=jax.ShapeDtypeStruct((V, B), jnp.float32),
    )(Wt, mean, b.reshape(1, V))
    return out_t.T


def kernel(context_words, embedding, W, b):
    B, C = context_words.shape
    V, D = embedding.shape
    idx_flat = context_words.reshape(B * C).astype(jnp.int32)
    mean = _mean_embed_sc(idx_flat, embedding, B, C, D)
    return _project_tc(mean, W.T, b, 4096)
